# baseline (speedup 1.0000x reference)
import jax
import jax.numpy as jnp
from jax.experimental import pallas as pl
from jax.experimental.pallas import tpu as pltpu

ISIZE = 224
PSIZE = 30
WIN = 40


def _ry_of(pos_ref, s):
    py = pos_ref[s, 0]
    return pl.multiple_of(jnp.minimum((py // 8) * 8, ISIZE - WIN), 8)


def _win_tile(pos_ref, pf_ref, s):
    py = pos_ref[s, 0]
    px = pos_ref[s, 1]
    dy = py - _ry_of(pos_ref, s)
    tile = pltpu.roll(pf_ref[0], px, axis=2)
    return pltpu.roll(tile, dy, axis=1)


def _make_kernel(B):
    def body(pos_ref, x_hbm, pf_ref, out_hbm, rbuf, wbuf, bsem, rsem, wsem):
        t = pl.program_id(0)

        def rd_copy(s):
            ry = _ry_of(pos_ref, s)
            return pltpu.make_async_copy(
                x_hbm.at[s, :, pl.ds(ry, WIN), :],
                rbuf.at[jax.lax.rem(s, 2)],
                rsem.at[jax.lax.rem(s, 2)],
            )

        def big_copy(s):
            return pltpu.make_async_copy(
                x_hbm.at[s], out_hbm.at[s], bsem.at[jax.lax.rem(s, 4)]
            )

        def wr_copy(s):
            ry = _ry_of(pos_ref, s)
            return pltpu.make_async_copy(
                wbuf.at[jax.lax.rem(s, 4)],
                out_hbm.at[s, :, pl.ds(ry, WIN), :],
                wsem.at[jax.lax.rem(s, 4)],
            )

        @pl.when(t < B)
        def _():
            big_copy(t).start()
            rd_copy(t).start()

        @pl.when((t >= 1) & (t <= B))
        def _():
            s = t - 1

            @pl.when(s >= 4)
            def _():
                wr_copy(s - 4).wait()

            rd_copy(s).wait()
            wbuf[jax.lax.rem(s, 4)] = (
                rbuf[jax.lax.rem(s, 2)] + _win_tile(pos_ref, pf_ref, s)
            )

        @pl.when((t >= 2) & (t - 2 < B))
        def _():
            s = t - 2
            big_copy(s).wait()
            wr_copy(s).start()

        @pl.when(t == B + 1)
        def _():
            for k in range(4):
                wr_copy(B - 4 + k).wait()

    return body


def kernel(x, patch, pos):
    B = x.shape[0]
    patch_pad = jnp.zeros((1, 3, WIN, ISIZE), dtype=patch.dtype)
    patch_pad = jax.lax.dynamic_update_slice(patch_pad, patch, (0, 0, 0, 0))
    grid_spec = pltpu.PrefetchScalarGridSpec(
        num_scalar_prefetch=1,
        grid=(B + 2,),
        in_specs=[
            pl.BlockSpec(memory_space=pl.ANY),
            pl.BlockSpec((1, 3, WIN, ISIZE), lambda t, pos_ref: (0, 0, 0, 0)),
        ],
        out_specs=pl.BlockSpec(memory_space=pl.ANY),
        scratch_shapes=[
            pltpu.VMEM((2, 3, WIN, ISIZE), jnp.float32),
            pltpu.VMEM((4, 3, WIN, ISIZE), jnp.float32),
            pltpu.SemaphoreType.DMA((4,)),
            pltpu.SemaphoreType.DMA((2,)),
            pltpu.SemaphoreType.DMA((4,)),
        ],
    )
    return pl.pallas_call(
        _make_kernel(B),
        grid_spec=grid_spec,
        out_shape=jax.ShapeDtypeStruct(x.shape, x.dtype),
    )(pos, x, patch_pad)


# speedup vs baseline: 13.3187x; 13.3187x over previous
import jax
import jax.numpy as jnp
from jax.experimental import pallas as pl
from jax.experimental.pallas import tpu as pltpu

ISIZE = 224
PSIZE = 30
WIN = 40
C = 4
K = 4
LAT = 2


def _win_tile(pos_ref, pf_ref, s):
    py = pos_ref[s, 0]
    px = pos_ref[s, 1]
    ry = pl.multiple_of(jnp.minimum((py // 8) * 8, ISIZE - WIN), 8)
    tile = pltpu.roll(pf_ref[0], px, axis=2)
    return ry, pltpu.roll(tile, py - ry, axis=1)


def _make_kernel(B):
    N = B // C

    def body(pos_ref, x_hbm, pf_ref, out_hbm, rbuf, rsem, wsem):
        t = pl.program_id(0)

        def rd_copy(c):
            k = jax.lax.rem(c, K)
            return pltpu.make_async_copy(
                x_hbm.at[pl.ds(c * C, C)], rbuf.at[pl.ds(k * C, C)], rsem.at[k]
            )

        def wr_copy(c):
            k = jax.lax.rem(c, K)
            return pltpu.make_async_copy(
                rbuf.at[pl.ds(k * C, C)], out_hbm.at[pl.ds(c * C, C)], wsem.at[k]
            )

        @pl.when(t < N)
        def _():
            @pl.when(t >= K)
            def _():
                wr_copy(t - K).wait()

            rd_copy(t).start()

        s = t - LAT

        @pl.when((s >= 0) & (s < N))
        def _():
            rd_copy(s).wait()
            k = jax.lax.rem(s, K)
            for i in range(C):
                b = s * C + i
                ry, tile = _win_tile(pos_ref, pf_ref, b)
                row = k * C + i
                rbuf[row, :, pl.ds(ry, WIN), :] = (
                    rbuf[row, :, pl.ds(ry, WIN), :] + tile
                )
            wr_copy(s).start()

        @pl.when(t == N + LAT - 1)
        def _():
            for j in range(K):
                wr_copy(N - K + j).wait()

    return body, N


def kernel(x, patch, pos):
    B = x.shape[0]
    patch_pad = jnp.zeros((1, 3, WIN, ISIZE), dtype=patch.dtype)
    patch_pad = jax.lax.dynamic_update_slice(patch_pad, patch, (0, 0, 0, 0))
    body, N = _make_kernel(B)
    grid_spec = pltpu.PrefetchScalarGridSpec(
        num_scalar_prefetch=1,
        grid=(N + LAT,),
        in_specs=[
            pl.BlockSpec(memory_space=pl.ANY),
            pl.BlockSpec((1, 3, WIN, ISIZE), lambda t, pos_ref: (0, 0, 0, 0)),
        ],
        out_specs=pl.BlockSpec(memory_space=pl.ANY),
        scratch_shapes=[
            pltpu.VMEM((K * C, 3, ISIZE, ISIZE), jnp.float32),
            pltpu.SemaphoreType.DMA((K,)),
            pltpu.SemaphoreType.DMA((K,)),
        ],
    )
    return pl.pallas_call(
        body,
        grid_spec=grid_spec,
        out_shape=jax.ShapeDtypeStruct(x.shape, x.dtype),
    )(pos, x, patch_pad)
